# baseline (speedup 1.0000x reference)
import jax
import jax.numpy as jnp
from jax.experimental import pallas as pl
from jax.experimental.pallas import tpu as pltpu

_GPB = 4


def _gcn_one(a, w1_ref, b1_ref, w2_ref, b2_ref, wl_ref, bl_ref):
    n = a.shape[0]
    deg = jnp.sum(a, axis=0)
    d = jnp.where(deg > 0, jax.lax.rsqrt(deg), 0.0)
    an = (d[:, None] * a * d[None, :]).astype(jnp.bfloat16)
    s1 = jnp.sum(an.astype(jnp.float32), axis=0)
    w1b = w1_ref[0].astype(jnp.bfloat16).astype(jnp.float32)
    t1 = s1[:, None] * w1b[None, :]
    h1 = jnp.maximum(t1 + b1_ref[0][None, :], 0.0)
    m = jnp.dot(h1.astype(jnp.bfloat16), w2_ref[...].astype(jnp.bfloat16),
                preferred_element_type=jnp.float32)
    t2 = jax.lax.dot_general(an, m.astype(jnp.bfloat16), (((0,), (0,)), ((), ())),
                             preferred_element_type=jnp.float32)
    h2 = jnp.maximum(t2 + b2_ref[0][None, :], 0.0)
    g = jnp.mean(h2, axis=0)
    gb = g.astype(jnp.bfloat16).astype(jnp.float32)
    wlb = wl_ref[...][:, 0].astype(jnp.bfloat16).astype(jnp.float32)
    return jnp.sum(gb * wlb) + bl_ref[0, 0]


def _gcn_kernel(a_ref, w1_ref, b1_ref, w2_ref, b2_ref, wl_ref, bl_ref, out_ref):
    logits = [_gcn_one(a_ref[i], w1_ref, b1_ref, w2_ref, b2_ref, wl_ref, bl_ref)
              for i in range(_GPB)]
    out_ref[...] = jnp.stack(logits).reshape(_GPB, 1, 1)


def kernel(A, W1, b1, W2, b2, Wl, bl):
    B, N, _ = A.shape
    H = W1.shape[1]
    out = pl.pallas_call(
        _gcn_kernel,
        grid=(B // _GPB,),
        in_specs=[
            pl.BlockSpec((_GPB, N, N), lambda b: (b, 0, 0)),
            pl.BlockSpec((1, H), lambda b: (0, 0)),
            pl.BlockSpec((1, H), lambda b: (0, 0)),
            pl.BlockSpec((H, H), lambda b: (0, 0)),
            pl.BlockSpec((1, H), lambda b: (0, 0)),
            pl.BlockSpec((H, 1), lambda b: (0, 0)),
            pl.BlockSpec((1, 1), lambda b: (0, 0)),
        ],
        out_specs=pl.BlockSpec((_GPB, 1, 1), lambda b: (b, 0, 0)),
        out_shape=jax.ShapeDtypeStruct((B, 1, 1), jnp.float32),
        compiler_params=pltpu.CompilerParams(
            dimension_semantics=("parallel",)),
    )(A, W1, b1.reshape(1, H), W2, b2.reshape(1, H), Wl, bl.reshape(1, 1))
    return out


# speedup vs baseline: 1.0799x; 1.0799x over previous
import jax
import jax.numpy as jnp
from jax.experimental import pallas as pl
from jax.experimental.pallas import tpu as pltpu

_GPB = 2


def _gcn_one(a, w1_ref, b1_ref, w2_ref, b2_ref, wl_ref, bl_ref):
    n = a.shape[0]
    deg = jnp.sum(a, axis=0)
    d = jnp.where(deg > 0, jax.lax.rsqrt(deg), 0.0)
    an = (d[:, None] * a * d[None, :]).astype(jnp.bfloat16)
    s1 = jnp.sum(an.astype(jnp.float32), axis=0)
    w1b = w1_ref[0].astype(jnp.bfloat16).astype(jnp.float32)
    t1 = s1[:, None] * w1b[None, :]
    h1 = jnp.maximum(t1 + b1_ref[0][None, :], 0.0)
    m = jnp.dot(h1.astype(jnp.bfloat16), w2_ref[...].astype(jnp.bfloat16),
                preferred_element_type=jnp.float32)
    t2t = jax.lax.dot_general(m.astype(jnp.bfloat16), an, (((0,), (0,)), ((), ())),
                              preferred_element_type=jnp.float32)
    t2 = t2t.T
    h2 = jnp.maximum(t2 + b2_ref[0][None, :], 0.0)
    g = jnp.mean(h2, axis=0)
    gb = g.astype(jnp.bfloat16).astype(jnp.float32)
    wlb = wl_ref[...][:, 0].astype(jnp.bfloat16).astype(jnp.float32)
    return jnp.sum(gb * wlb) + bl_ref[0, 0]


def _gcn_kernel(a_ref, w1_ref, b1_ref, w2_ref, b2_ref, wl_ref, bl_ref, out_ref):
    logits = [_gcn_one(a_ref[i], w1_ref, b1_ref, w2_ref, b2_ref, wl_ref, bl_ref)
              for i in range(_GPB)]
    out_ref[...] = jnp.stack(logits).reshape(_GPB, 1, 1)


def kernel(A, W1, b1, W2, b2, Wl, bl):
    B, N, _ = A.shape
    H = W1.shape[1]
    out = pl.pallas_call(
        _gcn_kernel,
        grid=(B // _GPB,),
        in_specs=[
            pl.BlockSpec((_GPB, N, N), lambda b: (b, 0, 0)),
            pl.BlockSpec((1, H), lambda b: (0, 0)),
            pl.BlockSpec((1, H), lambda b: (0, 0)),
            pl.BlockSpec((H, H), lambda b: (0, 0)),
            pl.BlockSpec((1, H), lambda b: (0, 0)),
            pl.BlockSpec((H, 1), lambda b: (0, 0)),
            pl.BlockSpec((1, 1), lambda b: (0, 0)),
        ],
        out_specs=pl.BlockSpec((_GPB, 1, 1), lambda b: (b, 0, 0)),
        out_shape=jax.ShapeDtypeStruct((B, 1, 1), jnp.float32),
        compiler_params=pltpu.CompilerParams(
            dimension_semantics=("parallel",)),
    )(A, W1, b1.reshape(1, H), W2, b2.reshape(1, H), Wl, bl.reshape(1, 1))
    return out


# speedup vs baseline: 1.0805x; 1.0005x over previous
import jax
import jax.numpy as jnp
from jax.experimental import pallas as pl
from jax.experimental.pallas import tpu as pltpu

_GPB = 2


def _gcn_one(a, w1_ref, b1_ref, w2_ref, b2_ref, wl_ref, bl_ref):
    n = a.shape[0]
    deg = jnp.sum(a, axis=0)
    d = jnp.where(deg > 0, jax.lax.rsqrt(deg), 0.0)
    an = (d[:, None] * a * d[None, :]).astype(jnp.bfloat16)
    s1 = jnp.sum(an.astype(jnp.float32), axis=0)
    w1b = w1_ref[0].astype(jnp.bfloat16).astype(jnp.float32)
    t1 = s1[:, None] * w1b[None, :]
    h1 = jnp.maximum(t1 + b1_ref[0][None, :], 0.0)
    m = jnp.dot(h1.astype(jnp.bfloat16), w2_ref[...].astype(jnp.bfloat16),
                preferred_element_type=jnp.float32)
    t2t = jax.lax.dot_general(m.astype(jnp.bfloat16), an, (((0,), (0,)), ((), ())),
                              preferred_element_type=jnp.float32)
    t2 = t2t.T
    h2 = jnp.maximum(t2 + b2_ref[0][None, :], 0.0)
    g = jnp.mean(h2, axis=0)
    gb = g.astype(jnp.bfloat16).astype(jnp.float32)
    wlb = wl_ref[...][:, 0].astype(jnp.bfloat16).astype(jnp.float32)
    return jnp.sum(gb * wlb) + bl_ref[0, 0]


def _gcn_kernel(a_ref, w1_ref, b1_ref, w2_ref, b2_ref, wl_ref, bl_ref, out_ref):
    logits = [_gcn_one(a_ref[i], w1_ref, b1_ref, w2_ref, b2_ref, wl_ref, bl_ref)
              for i in range(_GPB)]
    out_ref[...] = jnp.stack(logits).reshape(_GPB, 1, 1)


def kernel(A, W1, b1, W2, b2, Wl, bl):
    B, N, _ = A.shape
    H = W1.shape[1]
    out = pl.pallas_call(
        _gcn_kernel,
        grid=(B // _GPB,),
        in_specs=[
            pl.BlockSpec((_GPB, N, N), lambda b: (b, 0, 0)),
            pl.BlockSpec((1, H), lambda b: (0, 0)),
            pl.BlockSpec((1, H), lambda b: (0, 0)),
            pl.BlockSpec((H, H), lambda b: (0, 0)),
            pl.BlockSpec((1, H), lambda b: (0, 0)),
            pl.BlockSpec((H, 1), lambda b: (0, 0)),
            pl.BlockSpec((1, 1), lambda b: (0, 0)),
        ],
        out_specs=pl.BlockSpec((_GPB, 1, 1), lambda b: (b, 0, 0)),
        out_shape=jax.ShapeDtypeStruct((B, 1, 1), jnp.float32),
        compiler_params=pltpu.CompilerParams(
            dimension_semantics=("arbitrary",)),
    )(A, W1, b1.reshape(1, H), W2, b2.reshape(1, H), Wl, bl.reshape(1, 1))
    return out


# speedup vs baseline: 1.0968x; 1.0151x over previous
import jax
import jax.numpy as jnp
from jax.experimental import pallas as pl
from jax.experimental.pallas import tpu as pltpu

_GPB = 2


def _gcn_one(a, w1_ref, b1_ref, w2_ref, b2_ref, wl_ref, bl_ref):
    n = a.shape[0]
    deg = jnp.sum(a, axis=0)
    d = jnp.where(deg > 0, jax.lax.rsqrt(deg), 0.0)
    an = (d[:, None] * a * d[None, :]).astype(jnp.bfloat16)
    s1 = jnp.sum(an.astype(jnp.float32), axis=0)
    w1b = w1_ref[0].astype(jnp.bfloat16).astype(jnp.float32)
    t1 = s1[:, None] * w1b[None, :]
    h1 = jnp.maximum(t1 + b1_ref[0][None, :], 0.0)
    m = jnp.dot(h1.astype(jnp.bfloat16), w2_ref[...].astype(jnp.bfloat16),
                preferred_element_type=jnp.float32)
    t2t = jax.lax.dot_general(m.astype(jnp.bfloat16), an, (((0,), (0,)), ((), ())),
                              preferred_element_type=jnp.float32)
    h2t = jnp.maximum(t2t + b2_ref[0][:, None], 0.0)
    g = jnp.mean(h2t, axis=1)
    gb = g.astype(jnp.bfloat16).astype(jnp.float32)
    wlb = wl_ref[...][:, 0].astype(jnp.bfloat16).astype(jnp.float32)
    return jnp.sum(gb * wlb) + bl_ref[0, 0]


def _gcn_kernel(a_ref, w1_ref, b1_ref, w2_ref, b2_ref, wl_ref, bl_ref, out_ref):
    logits = [_gcn_one(a_ref[i], w1_ref, b1_ref, w2_ref, b2_ref, wl_ref, bl_ref)
              for i in range(_GPB)]
    out_ref[...] = jnp.stack(logits).reshape(_GPB, 1, 1)


def kernel(A, W1, b1, W2, b2, Wl, bl):
    B, N, _ = A.shape
    H = W1.shape[1]
    out = pl.pallas_call(
        _gcn_kernel,
        grid=(B // _GPB,),
        in_specs=[
            pl.BlockSpec((_GPB, N, N), lambda b: (b, 0, 0)),
            pl.BlockSpec((1, H), lambda b: (0, 0)),
            pl.BlockSpec((1, H), lambda b: (0, 0)),
            pl.BlockSpec((H, H), lambda b: (0, 0)),
            pl.BlockSpec((1, H), lambda b: (0, 0)),
            pl.BlockSpec((H, 1), lambda b: (0, 0)),
            pl.BlockSpec((1, 1), lambda b: (0, 0)),
        ],
        out_specs=pl.BlockSpec((_GPB, 1, 1), lambda b: (b, 0, 0)),
        out_shape=jax.ShapeDtypeStruct((B, 1, 1), jnp.float32),
        compiler_params=pltpu.CompilerParams(
            dimension_semantics=("arbitrary",)),
    )(A, W1, b1.reshape(1, H), W2, b2.reshape(1, H), Wl, bl.reshape(1, 1))
    return out
